# baseline (speedup 1.0000x reference)
import functools

import jax
import jax.numpy as jnp
from jax import lax
from jax.experimental import pallas as pl
from jax.experimental.pallas import tpu as pltpu
from jax.experimental.pallas import tpu_sc as plsc

_B, _N, _K, _H = 8, 2048, 16, 64
_RB1 = 512
_RB3 = 512
_PREC = jax.lax.Precision.HIGHEST


def _dot(a, b):
    return jax.lax.dot_general(
        a, b, (((a.ndim - 1,), (0,)), ((), ())),
        precision=_PREC, preferred_element_type=jnp.float32)


def _bdot(a, b):
    return jax.lax.dot_general(
        a, b, (((a.ndim - 1,), (0,)), ((), ())),
        preferred_element_type=jnp.float32)


def _k0_body(xp_ref, W1, b1, W2, b2, Wfc1, bfc1, Wd1, bd1, Wq, Wk, Wv,
             Wg1, E_ref, T_ref):
    x = xp_ref[0]
    h = _dot(jnp.maximum(_dot(x, W1[...]) + b1[...], 0.0), W2[...]) + b2[...]
    hh = _dot(h, Wfc1[...]) + bfc1[...]
    QG = _dot(hh, _dot(Wq[...], Wg1[...]))
    KG = _dot(hh, _dot(Wk[...], Wg1[...]))
    V = _dot(hh, Wv[...])
    A = _dot(x, Wd1[...]) + bd1[...]
    T = _dot(x, Wd1[...])
    E_ref[0] = jnp.concatenate([A, QG, h], axis=1)
    T_ref[0] = jnp.concatenate(
        [T, KG, V, jnp.zeros((x.shape[0], _H), jnp.float32)], axis=1)


def _k1_body(xp_ref, xt_ref, idx_ref):
    xb = xp_ref[0]
    xt = xt_ref[0]
    d = -2.0 * _dot(xb, xt)
    d = d + jnp.sum(xb * xb, axis=1, keepdims=True)
    d = d + jnp.sum(xt * xt, axis=0, keepdims=True)
    i32 = jax.lax.bitcast_convert_type(d, jnp.int32)
    key = jnp.where(i32 >= 0, i32, i32 ^ jnp.int32(0x7FFFFFFF))
    col = jax.lax.broadcasted_iota(jnp.int32, d.shape, 1)
    cur = (key & jnp.int32(-2048)) | col
    m = jnp.min(cur, axis=1, keepdims=True)
    cols = [m & jnp.int32(2047)]
    for _ in range(_K - 1):
        C = jnp.int32(-2147483648) - m - jnp.int32(1)
        mz = jnp.min(cur + C, axis=1, keepdims=True)
        m = mz - C
        cols.append(m & jnp.int32(2047))
    idx_ref[0] = jnp.concatenate(cols, axis=1)


def _k3_body(E_ref, G_ref, Wd2, bd2, Wg1, bg1, Wg2, bg2, Wfc2, bfc2,
             out_ref):
    E = E_ref[0]
    A = E[:, 0:_H]
    QG = E[:, _H:2 * _H]
    h = E[:, 2 * _H:3 * _H]
    Wd2v = Wd2[...]
    Wg1v = Wg1[...]
    Wcat = jnp.concatenate([Wd2v, _dot(Wd2v, Wg1v)], axis=1).astype(jnp.bfloat16)
    cvec = _dot(bd2[...], Wg1v) + bg1[...]
    Wg2v = Wg2[...].astype(jnp.bfloat16)
    bg2v = bg2[...]
    bd2v = bd2[...]
    M = jnp.full((_RB3, _H), -1e30, jnp.float32)
    S = jnp.zeros((_RB3, _H), jnp.float32)
    O = jnp.zeros((_RB3, _H), jnp.float32)
    for k in range(_K):
        Gk = G_ref[0, k]
        Tg = Gk[:, 0:_H]
        KGg = Gk[:, _H:2 * _H]
        Vg = Gk[:, 2 * _H:3 * _H]
        r1 = jnp.maximum(A - Tg, 0.0).astype(jnp.bfloat16)
        pe = _bdot(r1, Wcat)
        pos_enc = pe[:, 0:_H] + bd2v
        ah = jnp.maximum(QG - KGg + pe[:, _H:2 * _H] + cvec, 0.0)
        logit = (_bdot(ah.astype(jnp.bfloat16), Wg2v) + bg2v) * 0.125
        w = Vg + pos_enc
        Mn = jnp.maximum(M, logit)
        corr = jnp.exp(M - Mn)
        e = jnp.exp(logit - Mn)
        S = S * corr + e
        O = O * corr + e * w
        M = Mn
    res = _dot(O / S, Wfc2[...]) + bfc2[...] + h
    out_ref[0] = res


_SB = 4
_GROWS = _SB * _K * _N
_CHUNK = 128
_NW = 32


def _sc_gather_body(tbl_hbm, idx_hbm, out_hbm, idx_v, rows_v, sem):
    wid = lax.axis_index("s") * 2 + lax.axis_index("c")
    rows_per_worker = _GROWS // _NW
    base = wid * rows_per_worker

    def body(i, _):
        off = base + i * _CHUNK
        pltpu.sync_copy(idx_hbm.at[pl.ds(off, _CHUNK)], idx_v)
        pltpu.async_copy(tbl_hbm.at[idx_v], rows_v, sem).wait()
        pltpu.sync_copy(rows_v, out_hbm.at[pl.ds(off, _CHUNK)])
        return 0

    lax.fori_loop(0, rows_per_worker // _CHUNK, body, 0)


@functools.cache
def _sc_gather():
    return pl.kernel(
        _sc_gather_body,
        mesh=plsc.VectorSubcoreMesh(core_axis_name="c", subcore_axis_name="s"),
        out_type=jax.ShapeDtypeStruct((_GROWS, 4 * _H), jnp.float32),
        scratch_types=[
            pltpu.VMEM((_CHUNK,), jnp.int32),
            pltpu.VMEM((_CHUNK, 4 * _H), jnp.float32),
            pltpu.SemaphoreType.DMA,
        ],
    )


def kernel(x, W1, b1, W2, b2, Wfc1, bfc1, Wfc2, bfc2, Wd1, bd1, Wd2, bd2,
           Wg1, bg1, Wg2, bg2, Wq, Wk, Wv):
    f32 = jnp.float32
    B, N, _ = x.shape
    xp = jnp.concatenate([x, jnp.zeros((B, N, 5), f32)], axis=-1)
    xt = xp.transpose(0, 2, 1)
    W1p = jnp.concatenate([W1, jnp.zeros((5, _H), f32)], axis=0)
    Wd1p = jnp.concatenate([Wd1, jnp.zeros((5, _H), f32)], axis=0)
    b1r = b1.reshape(1, _H)
    b2r = b2.reshape(1, _H)
    bfc1r = bfc1.reshape(1, _H)
    bfc2r = bfc2.reshape(1, _H)
    bd1r = bd1.reshape(1, _H)
    bd2r = bd2.reshape(1, _H)
    bg1r = bg1.reshape(1, _H)
    bg2r = bg2.reshape(1, _H)

    wspec = lambda shape: pl.BlockSpec(shape, lambda *a: (0,) * len(shape))

    E, TBL = pl.pallas_call(
        _k0_body,
        grid=(B,),
        in_specs=[
            pl.BlockSpec((1, N, 8), lambda b: (b, 0, 0)),
            wspec((8, _H)), wspec((1, _H)), wspec((_H, _H)), wspec((1, _H)),
            wspec((_H, _H)), wspec((1, _H)),
            wspec((8, _H)), wspec((1, _H)),
            wspec((_H, _H)), wspec((_H, _H)), wspec((_H, _H)),
            wspec((_H, _H)),
        ],
        out_specs=(
            pl.BlockSpec((1, N, 3 * _H), lambda b: (b, 0, 0)),
            pl.BlockSpec((1, N, 4 * _H), lambda b: (b, 0, 0)),
        ),
        out_shape=(
            jax.ShapeDtypeStruct((B, N, 3 * _H), f32),
            jax.ShapeDtypeStruct((B, N, 4 * _H), f32),
        ),
    )(xp, W1p, b1r, W2, b2r, Wfc1, bfc1r, Wd1p, bd1r, Wq, Wk, Wv, Wg1)

    outs = []
    for s in range(0, B, _SB):
        xp_s = lax.slice_in_dim(xp, s, s + _SB, axis=0)
        xt_s = lax.slice_in_dim(xt, s, s + _SB, axis=0)

        idx_s = pl.pallas_call(
            _k1_body,
            grid=(_SB, N // _RB1),
            in_specs=[
                pl.BlockSpec((1, _RB1, 8), lambda g, r: (g, r, 0)),
                pl.BlockSpec((1, 8, N), lambda g, r: (g, 0, 0)),
            ],
            out_specs=pl.BlockSpec((1, _RB1, _K), lambda g, r: (g, r, 0)),
            out_shape=jax.ShapeDtypeStruct((_SB, N, _K), jnp.int32),
        )(xp_s, xt_s)

        base = ((jnp.arange(_SB, dtype=jnp.int32) + s) * N).reshape(_SB, 1, 1)
        gidx = (idx_s.transpose(0, 2, 1) + base).reshape(_GROWS)
        G = _sc_gather()(TBL.reshape(B * N, 4 * _H), gidx)
        G = G.reshape(_SB, _K, N, 4 * _H)

        E_s = lax.slice_in_dim(E, s, s + _SB, axis=0)
        res_s = pl.pallas_call(
            _k3_body,
            grid=(_SB, N // _RB3),
            in_specs=[
                pl.BlockSpec((1, _RB3, 3 * _H), lambda g, r: (g, r, 0)),
                pl.BlockSpec((1, _K, _RB3, 4 * _H), lambda g, r: (g, 0, r, 0)),
                wspec((_H, _H)), wspec((1, _H)), wspec((_H, _H)),
                wspec((1, _H)), wspec((_H, _H)), wspec((1, _H)),
                wspec((_H, _H)), wspec((1, _H)),
            ],
            out_specs=pl.BlockSpec((1, _RB3, _H), lambda g, r: (g, r, 0)),
            out_shape=jax.ShapeDtypeStruct((_SB, N, _H), f32),
        )(E_s, G, Wd2, bd2r, Wg1, bg1r, Wg2, bg2r, Wfc2, bfc2r)
        outs.append(res_s)
    return jnp.concatenate(outs, axis=0)


# speedup vs baseline: 1.0505x; 1.0505x over previous
import functools

import jax
import jax.numpy as jnp
from jax import lax
from jax.experimental import pallas as pl
from jax.experimental.pallas import tpu as pltpu
from jax.experimental.pallas import tpu_sc as plsc

_B, _N, _K, _H = 8, 2048, 16, 64
_RB1 = 512
_RB3 = 512
_PREC = jax.lax.Precision.HIGHEST


def _dot(a, b):
    return jax.lax.dot_general(
        a, b, (((a.ndim - 1,), (0,)), ((), ())),
        precision=_PREC, preferred_element_type=jnp.float32)


def _bdot(a, b):
    return jax.lax.dot_general(
        a, b, (((a.ndim - 1,), (0,)), ((), ())),
        preferred_element_type=jnp.float32)


def _k0_body(xp_ref, W1, b1, W2, b2, Wfc1, bfc1, Wd1, bd1, Wqg, Wkg, Wv,
             E_ref, T_ref):
    x = xp_ref[0]
    h = _dot(jnp.maximum(_dot(x, W1[...]) + b1[...], 0.0), W2[...]) + b2[...]
    hh = _dot(h, Wfc1[...]) + bfc1[...]
    QG = _dot(hh, Wqg[...])
    KG = _dot(hh, Wkg[...])
    V = _dot(hh, Wv[...])
    A = _dot(x, Wd1[...]) + bd1[...]
    T = _dot(x, Wd1[...])
    E_ref[0] = jnp.concatenate([A, QG, h], axis=1)
    T_ref[0] = jnp.concatenate(
        [T, KG, V, jnp.zeros((x.shape[0], _H), jnp.float32)], axis=1)


def _k1_body(xp_ref, xt_ref, idx_ref):
    xb = xp_ref[0]
    xt = xt_ref[0]
    d = -2.0 * _dot(xb, xt)
    d = d + jnp.sum(xb * xb, axis=1, keepdims=True)
    d = d + jnp.sum(xt * xt, axis=0, keepdims=True)
    i32 = jax.lax.bitcast_convert_type(d, jnp.int32)
    key = jnp.where(i32 >= 0, i32, i32 ^ jnp.int32(0x7FFFFFFF))
    col = jax.lax.broadcasted_iota(jnp.int32, d.shape, 1)
    cur = (key & jnp.int32(-2048)) | col
    m = jnp.min(cur, axis=1, keepdims=True)
    cols = [m & jnp.int32(2047)]
    for _ in range(_K - 1):
        C = jnp.int32(-2147483648) - m - jnp.int32(1)
        mz = jnp.min(cur + C, axis=1, keepdims=True)
        m = mz - C
        cols.append(m & jnp.int32(2047))
    idx_ref[0] = jnp.concatenate(cols, axis=1)


def _k3_body(E_ref, G_ref, Wcat_r, cvec_r, bd2, Wg2, bg2, Wfc2, bfc2,
             out_ref):
    E = E_ref[0]
    A = E[:, 0:_H]
    QG = E[:, _H:2 * _H]
    h = E[:, 2 * _H:3 * _H]
    Wcat = Wcat_r[...]
    cvec = cvec_r[...]
    Wg2v = Wg2[...].astype(jnp.bfloat16)
    bg2v = bg2[...]
    bd2v = bd2[...]
    M = jnp.full((_RB3, _H), -1e30, jnp.float32)
    S = jnp.zeros((_RB3, _H), jnp.float32)
    O = jnp.zeros((_RB3, _H), jnp.float32)
    for k in range(_K):
        Gk = G_ref[0, k]
        Tg = Gk[:, 0:_H]
        KGg = Gk[:, _H:2 * _H]
        Vg = Gk[:, 2 * _H:3 * _H]
        r1 = jnp.maximum(A - Tg, 0.0).astype(jnp.bfloat16)
        pe = _bdot(r1, Wcat)
        pos_enc = pe[:, 0:_H] + bd2v
        ah = jnp.maximum(QG - KGg + pe[:, _H:2 * _H] + cvec, 0.0)
        logit = (_bdot(ah.astype(jnp.bfloat16), Wg2v) + bg2v) * 0.125
        w = Vg + pos_enc
        Mn = jnp.maximum(M, logit)
        corr = jnp.exp(M - Mn)
        e = jnp.exp(logit - Mn)
        S = S * corr + e
        O = O * corr + e * w
        M = Mn
    res = _dot(O / S, Wfc2[...]) + bfc2[...] + h
    out_ref[0] = res


_SB = 2
_GROWS = _SB * _K * _N
_CHUNK = 128
_NW = 32


def _sc_gather_body(tbl_hbm, idx_hbm, out_hbm, idx_v, rows_v, sem):
    wid = lax.axis_index("s") * 2 + lax.axis_index("c")
    rows_per_worker = _GROWS // _NW
    base = wid * rows_per_worker

    def body(i, _):
        off = base + i * _CHUNK
        pltpu.sync_copy(idx_hbm.at[pl.ds(off, _CHUNK)], idx_v)
        pltpu.async_copy(tbl_hbm.at[idx_v], rows_v, sem).wait()
        pltpu.sync_copy(rows_v, out_hbm.at[pl.ds(off, _CHUNK)])
        return 0

    lax.fori_loop(0, rows_per_worker // _CHUNK, body, 0)


@functools.cache
def _sc_gather():
    return pl.kernel(
        _sc_gather_body,
        mesh=plsc.VectorSubcoreMesh(core_axis_name="c", subcore_axis_name="s"),
        out_type=jax.ShapeDtypeStruct((_GROWS, 4 * _H), jnp.float32),
        scratch_types=[
            pltpu.VMEM((_CHUNK,), jnp.int32),
            pltpu.VMEM((_CHUNK, 4 * _H), jnp.float32),
            pltpu.SemaphoreType.DMA,
        ],
    )


def kernel(x, W1, b1, W2, b2, Wfc1, bfc1, Wfc2, bfc2, Wd1, bd1, Wd2, bd2,
           Wg1, bg1, Wg2, bg2, Wq, Wk, Wv):
    f32 = jnp.float32
    B, N, _ = x.shape
    xp = jnp.concatenate([x, jnp.zeros((B, N, 5), f32)], axis=-1)
    xt = xp.transpose(0, 2, 1)
    W1p = jnp.concatenate([W1, jnp.zeros((5, _H), f32)], axis=0)
    Wd1p = jnp.concatenate([Wd1, jnp.zeros((5, _H), f32)], axis=0)
    b1r = b1.reshape(1, _H)
    b2r = b2.reshape(1, _H)
    bfc1r = bfc1.reshape(1, _H)
    bfc2r = bfc2.reshape(1, _H)
    bd1r = bd1.reshape(1, _H)
    bd2r = bd2.reshape(1, _H)
    bg1r = bg1.reshape(1, _H)
    bg2r = bg2.reshape(1, _H)

    wspec = lambda shape: pl.BlockSpec(shape, lambda *a: (0,) * len(shape))

    Wqg = Wq @ Wg1
    Wkg = Wk @ Wg1
    Wcat = jnp.concatenate([Wd2, Wd2 @ Wg1], axis=1).astype(jnp.bfloat16)
    cvec = (bd2.reshape(1, _H) @ Wg1) + bg1.reshape(1, _H)

    E, TBL = pl.pallas_call(
        _k0_body,
        grid=(B,),
        in_specs=[
            pl.BlockSpec((1, N, 8), lambda b: (b, 0, 0)),
            wspec((8, _H)), wspec((1, _H)), wspec((_H, _H)), wspec((1, _H)),
            wspec((_H, _H)), wspec((1, _H)),
            wspec((8, _H)), wspec((1, _H)),
            wspec((_H, _H)), wspec((_H, _H)), wspec((_H, _H)),
        ],
        out_specs=(
            pl.BlockSpec((1, N, 3 * _H), lambda b: (b, 0, 0)),
            pl.BlockSpec((1, N, 4 * _H), lambda b: (b, 0, 0)),
        ),
        out_shape=(
            jax.ShapeDtypeStruct((B, N, 3 * _H), f32),
            jax.ShapeDtypeStruct((B, N, 4 * _H), f32),
        ),
    )(xp, W1p, b1r, W2, b2r, Wfc1, bfc1r, Wd1p, bd1r, Wqg, Wkg, Wv)

    outs = []
    for s in range(0, B, _SB):
        xp_s = lax.slice_in_dim(xp, s, s + _SB, axis=0)
        xt_s = lax.slice_in_dim(xt, s, s + _SB, axis=0)

        idx_s = pl.pallas_call(
            _k1_body,
            grid=(_SB, N // _RB1),
            in_specs=[
                pl.BlockSpec((1, _RB1, 8), lambda g, r: (g, r, 0)),
                pl.BlockSpec((1, 8, N), lambda g, r: (g, 0, 0)),
            ],
            out_specs=pl.BlockSpec((1, _RB1, _K), lambda g, r: (g, r, 0)),
            out_shape=jax.ShapeDtypeStruct((_SB, N, _K), jnp.int32),
        )(xp_s, xt_s)

        base = ((jnp.arange(_SB, dtype=jnp.int32) + s) * N).reshape(_SB, 1, 1)
        gidx = (idx_s.transpose(0, 2, 1) + base).reshape(_GROWS)
        G = _sc_gather()(TBL.reshape(B * N, 4 * _H), gidx)
        G = G.reshape(_SB, _K, N, 4 * _H)

        E_s = lax.slice_in_dim(E, s, s + _SB, axis=0)
        res_s = pl.pallas_call(
            _k3_body,
            grid=(_SB, N // _RB3),
            in_specs=[
                pl.BlockSpec((1, _RB3, 3 * _H), lambda g, r: (g, r, 0)),
                pl.BlockSpec((1, _K, _RB3, 4 * _H), lambda g, r: (g, 0, r, 0)),
                wspec((_H, 2 * _H)), wspec((1, _H)), wspec((1, _H)),
                wspec((_H, _H)), wspec((1, _H)),
                wspec((_H, _H)), wspec((1, _H)),
            ],
            out_specs=pl.BlockSpec((1, _RB3, _H), lambda g, r: (g, r, 0)),
            out_shape=jax.ShapeDtypeStruct((_SB, N, _H), f32),
        )(E_s, G, Wcat, cvec, bd2r, Wg2, bg2r, Wfc2, bfc2r)
        outs.append(res_s)
    return jnp.concatenate(outs, axis=0)
